# Initial kernel scaffold; baseline (speedup 1.0000x reference)
import jax
import jax.numpy as jnp
from jax.experimental import pallas as pl


def kernel(x, weight, bias):
    raise NotImplementedError("write your pallas kernel here")


# speedup vs baseline: 1.1206x; 1.1206x over previous
import functools

import jax
import jax.numpy as jnp
from jax import lax
from jax.experimental import pallas as pl
from jax.experimental.pallas import tpu as pltpu
from jax.experimental.pallas import tpu_sc as plsc

F = 26
FIELD_DIM = 100000
B = 16384
NC, NS, L = 2, 16, 16
NW = NC * NS
BPW = B // NW
NCHUNK = 4
CW = BPW // NCHUNK


def _body(xt_hbm, w_hbm, bias_hbm, out_hbm, xt_v, idx_v, vals_v, acc_v, sem):
    wid = lax.axis_index("s") * NC + lax.axis_index("c")
    base = wid * BPW

    pltpu.sync_copy(xt_hbm.at[:, pl.ds(base, BPW)], xt_v)

    pltpu.sync_copy(bias_hbm, vals_v.at[0, pl.ds(0, L)])
    bv = vals_v[0, pl.ds(0, L)]
    for k in range(BPW // L):
        acc_v[pl.ds(k * L, L)] = bv

    def field_step(f, _):
        off = f * FIELD_DIM
        for j in range(NCHUNK):
            for k in range(CW // L):
                x16 = xt_v[f, pl.ds(j * CW + k * L, L)]
                idx_v[j, pl.ds(k * L, L)] = x16 + off
        copies = [
            pltpu.async_copy(w_hbm.at[idx_v.at[j]], vals_v.at[j], sem)
            for j in range(NCHUNK)
        ]
        for c in copies:
            c.wait()
        for j in range(NCHUNK):
            for k in range(CW // L):
                plsc.addupdate(
                    acc_v.at[pl.ds(j * CW + k * L, L)],
                    vals_v[j, pl.ds(k * L, L)],
                )
        return _

    lax.fori_loop(0, F, field_step, None)

    pltpu.sync_copy(acc_v, out_hbm.at[pl.ds(base, BPW)])


@jax.jit
def _features_linear(xt, w, bias16):
    mesh = plsc.VectorSubcoreMesh(
        core_axis_name="c", subcore_axis_name="s", num_cores=NC, num_subcores=NS
    )
    run = pl.kernel(
        _body,
        out_type=jax.ShapeDtypeStruct((B,), jnp.float32),
        mesh=mesh,
        scratch_types=[
            pltpu.VMEM((F, BPW), jnp.int32),
            pltpu.VMEM((NCHUNK, CW), jnp.int32),
            pltpu.VMEM((NCHUNK, CW), jnp.float32),
            pltpu.VMEM((BPW,), jnp.float32),
            pltpu.SemaphoreType.DMA,
        ],
    )
    return run(xt, w, bias16)


def kernel(x, weight, bias):
    xt = x.T
    w = weight.reshape(-1)
    bias16 = jnp.broadcast_to(bias, (L,)).astype(jnp.float32)
    out = _features_linear(xt, w, bias16)
    return out.reshape(B, 1)


# speedup vs baseline: 1.2370x; 1.1039x over previous
import jax
import jax.numpy as jnp
from jax import lax
from jax.experimental import pallas as pl
from jax.experimental.pallas import tpu as pltpu
from jax.experimental.pallas import tpu_sc as plsc

F = 26
FIELD_DIM = 100000
FIELD_PAD = 100096
B = 16384
NC, NS, L = 2, 16, 16
NF = F // NC
NH = 8
HQ = B // NH
RB = B // NS
RQ = 512
NR = RB // RQ


def _sc_body(xt_hbm, w_hbm, part_hbm, sub_v, x_v, val_v, red_v, o_v, acc_sh, sem):
    sc = lax.axis_index("c")
    sid = lax.axis_index("s")

    @pl.when(sid < NF)
    def _gather_phase():
        f = sc * NF + sid
        pltpu.sync_copy(
            w_hbm.at[pl.ds(f * FIELD_DIM, FIELD_DIM)], sub_v.at[pl.ds(0, FIELD_DIM)]
        )
        for h in range(NH):
            pltpu.sync_copy(xt_hbm.at[f, pl.ds(h * HQ, HQ)], x_v)

            @pl.loop(0, HQ, step=L, unroll=4)
            def _g(i):
                xi = x_v[pl.ds(i, L)]
                val_v[pl.ds(i, L)] = plsc.load_gather(sub_v, [xi])

            pltpu.sync_copy(val_v, acc_sh.at[pl.ds(sid * B + h * HQ, HQ)])

    plsc.subcore_barrier()

    for r in range(NR):
        base = sid * RB + r * RQ
        for row in range(NF):
            pltpu.sync_copy(
                acc_sh.at[pl.ds(row * B + base, RQ)],
                red_v.at[pl.ds(row * RQ, RQ)],
            )

        @pl.loop(0, RQ, step=L, unroll=2)
        def _r(c):
            s = red_v[pl.ds(c, L)]
            for row in range(1, NF):
                s = s + red_v[pl.ds(row * RQ + c, L)]
            o_v[pl.ds(r * RQ + c, L)] = s

    pltpu.sync_copy(o_v, part_hbm.at[pl.ds(sc * B + sid * RB, RB)])


def _tc_body(p_ref, b_ref, o_ref):
    o_ref[...] = p_ref[0] + p_ref[1] + b_ref[...]


@jax.jit
def _features_linear(xt, w, bias_row):
    mesh = plsc.VectorSubcoreMesh(
        core_axis_name="c", subcore_axis_name="s", num_cores=NC, num_subcores=NS
    )
    part = pl.kernel(
        _sc_body,
        out_type=jax.ShapeDtypeStruct((NC * B,), jnp.float32),
        mesh=mesh,
        compiler_params=pltpu.CompilerParams(needs_layout_passes=False),
        scratch_types=[
            pltpu.VMEM((FIELD_PAD,), jnp.float32),
            pltpu.VMEM((HQ,), jnp.int32),
            pltpu.VMEM((HQ,), jnp.float32),
            pltpu.VMEM((NF * RQ,), jnp.float32),
            pltpu.VMEM((RB,), jnp.float32),
            pltpu.VMEM_SHARED((NF * B,), jnp.float32),
            pltpu.SemaphoreType.DMA,
        ],
    )(xt, w)
    out = pl.pallas_call(
        _tc_body,
        out_shape=jax.ShapeDtypeStruct((B // 128, 128), jnp.float32),
    )(part.reshape(NC, B // 128, 128), bias_row)
    return out


def kernel(x, weight, bias):
    xt = x.T
    w = weight.reshape(-1)
    bias_row = jnp.broadcast_to(bias.astype(jnp.float32), (1, 128))
    out = _features_linear(xt, w, bias_row)
    return out.reshape(B, 1)


# speedup vs baseline: 1.3012x; 1.0519x over previous
import jax
import jax.numpy as jnp
from jax import lax
from jax.experimental import pallas as pl
from jax.experimental.pallas import tpu as pltpu
from jax.experimental.pallas import tpu_sc as plsc

F = 26
FIELD_DIM = 100000
FIELD_PAD = 100096
B = 16384
NC, NS, L = 2, 16, 16
NF = F // NC
NH = 8
HQ = B // NH
RB = B // NS
RQ = 512
NR = RB // RQ


def _sc_body(
    xt_hbm, w_hbm, part_hbm, sub_v, x_v, val_v, red_v, o_v, acc_sh, sem, sem_s,
    sem_x0, sem_x1
):
    sc = lax.axis_index("c")
    sid = lax.axis_index("s")

    @pl.when(sid < NF)
    def _gather_phase():
        f = sc * NF + sid
        sems = [sem_x0, sem_x1]
        sub_c = pltpu.async_copy(
            w_hbm.at[pl.ds(f * FIELD_DIM, FIELD_DIM)],
            sub_v.at[pl.ds(0, FIELD_DIM)],
            sem_s,
        )
        cps = [None, None]
        cps[0] = pltpu.async_copy(
            xt_hbm.at[f, pl.ds(0, HQ)], x_v.at[pl.ds(0, HQ)], sems[0]
        )
        sub_c.wait()
        for h in range(NH):
            p = h % 2
            if h + 1 < NH:
                q = (h + 1) % 2
                cps[q] = pltpu.async_copy(
                    xt_hbm.at[f, pl.ds((h + 1) * HQ, HQ)],
                    x_v.at[pl.ds(q * HQ, HQ)],
                    sems[q],
                )
            cps[p].wait()

            @pl.loop(0, HQ, step=L, unroll=4)
            def _g(i):
                xi = x_v[pl.ds(p * HQ + i, L)]
                val_v[pl.ds(i, L)] = plsc.load_gather(sub_v, [xi])

            pltpu.sync_copy(val_v, acc_sh.at[pl.ds(sid * B + h * HQ, HQ)])

    plsc.subcore_barrier()

    for r in range(NR):
        base = sid * RB + r * RQ
        copies = [
            pltpu.async_copy(
                acc_sh.at[pl.ds(row * B + base, RQ)],
                red_v.at[pl.ds(row * RQ, RQ)],
                sem,
            )
            for row in range(NF)
        ]
        for c in copies:
            c.wait()

        @pl.loop(0, RQ, step=L, unroll=2)
        def _r(c):
            s = red_v[pl.ds(c, L)]
            for row in range(1, NF):
                s = s + red_v[pl.ds(row * RQ + c, L)]
            o_v[pl.ds(r * RQ + c, L)] = s

    pltpu.sync_copy(o_v, part_hbm.at[pl.ds(sc * B + sid * RB, RB)])


def _tc_body(p_ref, b_ref, o_ref):
    o_ref[...] = p_ref[0] + p_ref[1] + b_ref[...]


@jax.jit
def _features_linear(xt, w, bias_row):
    mesh = plsc.VectorSubcoreMesh(
        core_axis_name="c", subcore_axis_name="s", num_cores=NC, num_subcores=NS
    )
    part = pl.kernel(
        _sc_body,
        out_type=jax.ShapeDtypeStruct((NC * B,), jnp.float32),
        mesh=mesh,
        compiler_params=pltpu.CompilerParams(needs_layout_passes=False),
        scratch_types=[
            pltpu.VMEM((FIELD_PAD,), jnp.float32),
            pltpu.VMEM((2 * HQ,), jnp.int32),
            pltpu.VMEM((HQ,), jnp.float32),
            pltpu.VMEM((NF * RQ,), jnp.float32),
            pltpu.VMEM((RB,), jnp.float32),
            pltpu.VMEM_SHARED((NF * B,), jnp.float32),
            pltpu.SemaphoreType.DMA,
            pltpu.SemaphoreType.DMA,
            pltpu.SemaphoreType.DMA,
            pltpu.SemaphoreType.DMA,
        ],
    )(xt, w)
    out = pl.pallas_call(
        _tc_body,
        out_shape=jax.ShapeDtypeStruct((B // 128, 128), jnp.float32),
    )(part.reshape(NC, B // 128, 128), bias_row)
    return out


def kernel(x, weight, bias):
    xt = x.T
    w = weight.reshape(-1)
    bias_row = jnp.broadcast_to(bias.astype(jnp.float32), (1, 128))
    out = _features_linear(xt, w, bias_row)
    return out.reshape(B, 1)


# speedup vs baseline: 1.6673x; 1.2814x over previous
import jax
import jax.numpy as jnp
from jax import lax
from jax.experimental import pallas as pl
from jax.experimental.pallas import tpu as pltpu
from jax.experimental.pallas import tpu_sc as plsc

F = 26
FIELD_DIM = 100000
FIELD_PAD = 100096
B = 16384
NC, NS, L = 2, 16, 16
KC = 2
NFK = F // KC
NFA = 7
NH = 8
HQ = B // NH
RB = B // NS
RQ = 512
NR = RB // RQ


def _mk_sc_body(fb):
    def _sc_body(
        xt_hbm, wk_hbm, part_hbm, sub_v, x_v, val_v, red_v, o_v, acc_sh, sem,
        sem_s, sem_x0, sem_x1
    ):
        sc = lax.axis_index("c")
        sid = lax.axis_index("s")
        j = sc * NFA + sid

        @pl.when(sid < NFA - sc)
        def _gather_phase():
            f = fb + j
            sems = [sem_x0, sem_x1]
            sub_c = pltpu.async_copy(
                wk_hbm.at[pl.ds(j * FIELD_DIM, FIELD_DIM)],
                sub_v.at[pl.ds(0, FIELD_DIM)],
                sem_s,
            )
            cps = [None, None]
            cps[0] = pltpu.async_copy(
                xt_hbm.at[f, pl.ds(0, HQ)], x_v.at[pl.ds(0, HQ)], sems[0]
            )
            sub_c.wait()
            for h in range(NH):
                p = h % 2
                if h + 1 < NH:
                    q = (h + 1) % 2
                    cps[q] = pltpu.async_copy(
                        xt_hbm.at[f, pl.ds((h + 1) * HQ, HQ)],
                        x_v.at[pl.ds(q * HQ, HQ)],
                        sems[q],
                    )
                cps[p].wait()

                @pl.loop(0, HQ, step=L, unroll=4)
                def _g(i):
                    xi = x_v[pl.ds(p * HQ + i, L)]
                    val_v[pl.ds(i, L)] = plsc.load_gather(sub_v, [xi])

                pltpu.sync_copy(val_v, acc_sh.at[pl.ds(sid * B + h * HQ, HQ)])

        @pl.when((sc == 1) & (sid == NFA - 1))
        def _zero_unused_row():
            @pl.loop(0, HQ, step=L)
            def _z(i):
                val_v[pl.ds(i, L)] = jnp.zeros((L,), jnp.float32)

            for h in range(NH):
                pltpu.sync_copy(val_v, acc_sh.at[pl.ds(sid * B + h * HQ, HQ)])

        plsc.subcore_barrier()

        for r in range(NR):
            base = sid * RB + r * RQ
            copies = [
                pltpu.async_copy(
                    acc_sh.at[pl.ds(row * B + base, RQ)],
                    red_v.at[pl.ds(row * RQ, RQ)],
                    sem,
                )
                for row in range(NFA)
            ]
            for c in copies:
                c.wait()

            @pl.loop(0, RQ, step=L, unroll=2)
            def _r(c):
                s = red_v[pl.ds(c, L)]
                for row in range(1, NFA):
                    s = s + red_v[pl.ds(row * RQ + c, L)]
                o_v[pl.ds(r * RQ + c, L)] = s

        pltpu.sync_copy(o_v, part_hbm.at[pl.ds(sc * B + sid * RB, RB)])

    return _sc_body


def _tc_body(p0_ref, p1_ref, b_ref, o_ref):
    o_ref[...] = (
        p0_ref[0] + p0_ref[1] + p1_ref[0] + p1_ref[1] + b_ref[...]
    )


@jax.jit
def _features_linear(xt, w0, w1, bias_row):
    mesh = plsc.VectorSubcoreMesh(
        core_axis_name="c", subcore_axis_name="s", num_cores=NC, num_subcores=NS
    )
    scratch = [
        pltpu.VMEM((FIELD_PAD,), jnp.float32),
        pltpu.VMEM((2 * HQ,), jnp.int32),
        pltpu.VMEM((HQ,), jnp.float32),
        pltpu.VMEM((NFA * RQ,), jnp.float32),
        pltpu.VMEM((RB,), jnp.float32),
        pltpu.VMEM_SHARED((NFA * B,), jnp.float32),
        pltpu.SemaphoreType.DMA,
        pltpu.SemaphoreType.DMA,
        pltpu.SemaphoreType.DMA,
        pltpu.SemaphoreType.DMA,
    ]
    parts = []
    for k, wk in enumerate((w0, w1)):
        parts.append(
            pl.kernel(
                _mk_sc_body(k * NFK),
                out_type=jax.ShapeDtypeStruct((NC * B,), jnp.float32),
                mesh=mesh,
                compiler_params=pltpu.CompilerParams(needs_layout_passes=False),
                scratch_types=scratch,
                name=f"sc_fields_{k}",
            )(xt, wk)
        )
    out = pl.pallas_call(
        _tc_body,
        out_shape=jax.ShapeDtypeStruct((B // 128, 128), jnp.float32),
    )(
        parts[0].reshape(NC, B // 128, 128),
        parts[1].reshape(NC, B // 128, 128),
        bias_row,
    )
    return out


def kernel(x, weight, bias):
    xt = x.T
    w0 = weight[: NFK * FIELD_DIM].reshape(-1)
    w1 = weight[NFK * FIELD_DIM :].reshape(-1)
    bias_row = jnp.broadcast_to(bias.astype(jnp.float32), (1, 128))
    out = _features_linear(xt, w0, w1, bias_row)
    return out.reshape(B, 1)
